# Initial kernel scaffold; baseline (speedup 1.0000x reference)
import jax
import jax.numpy as jnp
from jax.experimental import pallas as pl


def kernel(x, pos, batch_size, table, W, b):
    raise NotImplementedError("write your pallas kernel here")


# speedup vs baseline: 9.6886x; 9.6886x over previous
import functools

import jax
import jax.numpy as jnp
from jax import lax
from jax.experimental import pallas as pl
from jax.experimental.pallas import tpu as pltpu
from jax.experimental.pallas import tpu_sc as plsc

_D = 300
_CONC = 219
_B = 4096
_L = 200
_ROWS_BLK = 4096
_NC, _NS = 2, 16
_NW = _NC * _NS
_RPW = _B // _NW
_GRP = _RPW // 16


def _tv_body(tab_ref, w_ref, out_ref):
    out_ref[...] = jnp.sum(tab_ref[...] * w_ref[0, :][None, :], axis=1)


def _make_tv(n_blocks):
    return pl.pallas_call(
        _tv_body,
        grid=(n_blocks,),
        in_specs=[
            pl.BlockSpec((_ROWS_BLK, _D), lambda i: (i, 0)),
            pl.BlockSpec((1, _D), lambda i: (0, 0)),
        ],
        out_specs=pl.BlockSpec((_ROWS_BLK,), lambda i: (i,)),
        out_shape=jax.ShapeDtypeStruct((n_blocks * _ROWS_BLK,), jnp.float32),
    )


def _pd_body(pos_ref, w_ref, b_ref, out_ref):
    out_ref[0, :] = (
        jnp.sum(pos_ref[...] * w_ref[0, :][None, :], axis=1) + b_ref[0, 0]
    )


def _make_pd():
    return pl.pallas_call(
        _pd_body,
        in_specs=[
            pl.BlockSpec((_B, _CONC), lambda: (0, 0)),
            pl.BlockSpec((1, _CONC), lambda: (0, 0)),
            pl.BlockSpec((1, 1), lambda: (0, 0)),
        ],
        out_specs=pl.BlockSpec((1, _B), lambda: (0, 0)),
        out_shape=jax.ShapeDtypeStruct((1, _B), jnp.float32),
    )


def _make_sc(vpad):
    mesh = plsc.VectorSubcoreMesh(
        core_axis_name="c", subcore_axis_name="s",
        num_cores=_NC, num_subcores=_NS,
    )

    @functools.partial(
        pl.kernel,
        mesh=mesh,
        compiler_params=pltpu.CompilerParams(needs_layout_passes=False),
        out_type=jax.ShapeDtypeStruct((_B,), jnp.float32),
        scratch_types=[
            pltpu.VMEM((vpad,), jnp.float32),
            pltpu.VMEM((_L, _RPW), jnp.int32),
            pltpu.VMEM((_RPW,), jnp.float32),
            pltpu.VMEM((_RPW,), jnp.float32),
        ],
    )
    def sc_fn(xt_hbm, tv_hbm, posb_hbm, out_hbm, tv_v, idx_v, pd_v, out_v):
        wid = lax.axis_index("s") * _NC + lax.axis_index("c")
        base = wid * _RPW
        pltpu.sync_copy(tv_hbm, tv_v)
        pltpu.sync_copy(xt_hbm.at[wid], idx_v)
        pltpu.sync_copy(posb_hbm.at[pl.ds(base, _RPW)], pd_v)

        def l_body(l, accs):
            new = []
            for g in range(_GRP):
                iv = idx_v[l, pl.ds(g * 16, 16)]
                new.append(accs[g] + plsc.load_gather(tv_v, [iv]))
            return tuple(new)

        zero = jnp.zeros((16,), jnp.float32)
        accs = lax.fori_loop(0, _L, l_body, (zero,) * _GRP)
        for g in range(_GRP):
            out_v[pl.ds(g * 16, 16)] = (
                accs[g] * jnp.float32(1.0 / _L) + pd_v[pl.ds(g * 16, 16)]
            )
        pltpu.sync_copy(out_v, out_hbm.at[pl.ds(base, _RPW)])

    return sc_fn


def kernel(x, pos, batch_size, table, W, b):
    B, L = x.shape
    rows, D = table.shape
    conc = pos.shape[2]
    assert (B, L, D, conc) == (_B, _L, _D, _CONC)

    n_blocks = pl.cdiv(rows, _ROWS_BLK)
    vpad = n_blocks * _ROWS_BLK

    w_emb = W[:D, 0].reshape(1, D)
    w_pos = W[D:, 0].reshape(1, conc)

    tv = _make_tv(n_blocks)(table, w_emb)
    posb = _make_pd()(pos.reshape(B, conc), w_pos, b.reshape(1, 1)).reshape(B)

    xt = x.astype(jnp.int32).reshape(_NW, _RPW, L).transpose(0, 2, 1)

    return _make_sc(vpad)(xt, tv, posb)
